# baseline (speedup 1.0000x reference)
import functools

import jax
import jax.numpy as jnp
from jax import lax
from jax.experimental import pallas as pl
from jax.experimental.pallas import tpu as pltpu, tpu_sc as plsc

N = 16384
C = 16
N_BINS = 50
EMB_DIM = 64
GROUP_SIZES = (6, 3, 3, 4)
GROUP_STARTS = (0, 6, 9, 12)
NQ = N_BINS - 1

BCH = 8192
OCH = 2048

NW = 32
RPW = N // NW
RCH = 128
NCHK = RPW // RCH


def _bins_kernel(keys_ref, x_ref, j_ref, lw_ref, hw_ref, goff_ref, bins_ref):
    j2 = j_ref[:]
    top = jnp.uint32(0x80000000)

    def bit_step(i, p):
        shift = (jnp.int32(31) - i).astype(jnp.uint32)
        trial = p | (jnp.uint32(1) << shift)

        def chunk_step(c, cnt):
            kb = keys_ref[:, pl.ds(c * BCH, BCH)]
            lt = kb[None, :, :] < trial[:, :, None]
            return cnt + jnp.sum(lt.astype(jnp.int32), axis=-1)

        cnt = jax.lax.fori_loop(
            0, N // BCH, chunk_step, jnp.zeros((NQ, C), jnp.int32))
        return jnp.where(cnt <= j2, trial, p)

    p = jax.lax.fori_loop(0, 32, bit_step, jnp.zeros((NQ, C), jnp.uint32))

    imax = jnp.int32(0x7FFFFFFF)

    def succ_step(c, carry):
        cle, mgt = carry
        kb = keys_ref[:, pl.ds(c * BCH, BCH)]
        kb3 = kb[None, :, :]
        le = kb3 <= p[:, :, None]
        cle = cle + jnp.sum(le.astype(jnp.int32), axis=-1)
        kb3_i = jax.lax.bitcast_convert_type(kb3 ^ top, jnp.int32)
        gtv = jnp.where(le, imax, kb3_i)
        mgt = jnp.minimum(mgt, jnp.min(gtv, axis=-1))
        return cle, mgt

    cle, mgt = jax.lax.fori_loop(
        0, N // BCH, succ_step,
        (jnp.zeros((NQ, C), jnp.int32), jnp.full((NQ, C), imax)))
    mgt_u = jax.lax.bitcast_convert_type(mgt, jnp.uint32) ^ top
    p_hi = jnp.where(cle >= j2 + 2, p, mgt_u)

    def unmap(k):
        u = jnp.where(k >= top, k & jnp.uint32(0x7FFFFFFF), ~k)
        return jax.lax.bitcast_convert_type(u, jnp.float32)

    edges = unmap(p) * lw_ref[:] + unmap(p_hi) * hw_ref[:]

    goff = goff_ref[:]

    def out_step(c, carry):
        xc = x_ref[:, pl.ds(c * OCH, OCH)]
        le = edges[:, :, None] <= xc[None, :, :]
        bins = jnp.sum(le.astype(jnp.int32), axis=0)
        bins_ref[:, pl.ds(c * OCH, OCH)] = bins + goff
        return carry

    jax.lax.fori_loop(0, N // OCH, out_step, jnp.int32(0))


def _make_sc_pool():
    mesh = plsc.VectorSubcoreMesh(core_axis_name="c", subcore_axis_name="s")

    @functools.partial(
        pl.kernel, mesh=mesh,
        out_type=jax.ShapeDtypeStruct((4, N, 2 * EMB_DIM), jnp.float32),
        scratch_types=[
            pltpu.VMEM((4, RCH), jnp.int32),
            pltpu.VMEM((RCH,), jnp.int32),
            pltpu.VMEM((RCH, 2 * EMB_DIM), jnp.float32),
            pltpu.VMEM_SHARED((16 * 4 * RCH, 2 * EMB_DIM), jnp.float32),
            pltpu.SemaphoreType.DMA,
        ],
    )
    def pool(table_hbm, bins_hbm, sidx_hbm, out_hbm,
             sidx_v, idx_v, gbuf, shacc, sem):
        cid = lax.axis_index("c")
        sid = lax.axis_index("s")
        wid = sid * 2 + cid
        pltpu.sync_copy(sidx_hbm.at[sid], sidx_v)
        for chunk in range(NCHK):
            base = wid * RPW + chunk * RCH
            for g in range(4):
                s0 = GROUP_STARTS[g]
                reg = (sid * 4 + g) * RCH
                for j in range(GROUP_SIZES[g]):
                    col = s0 + j
                    pltpu.sync_copy(bins_hbm.at[col, pl.ds(base, RCH)], idx_v)
                    if j == 0:
                        pltpu.async_copy(table_hbm.at[idx_v], gbuf, sem).wait()
                        pltpu.sync_copy(gbuf, shacc.at[pl.ds(reg, RCH)])
                    else:
                        pltpu.async_copy(table_hbm.at[idx_v], gbuf, sem).wait()
                        pltpu.sync_copy(gbuf, shacc.at[sidx_v.at[g]],
                                        add=True)
                pltpu.sync_copy(shacc.at[pl.ds(reg, RCH)],
                                out_hbm.at[g, pl.ds(base, RCH)])

    return pool


@functools.partial(jax.jit)
def kernel(features, emb_global, emb_hour, emb_session, emb_impression):
    xt = features.T
    u = jax.lax.bitcast_convert_type(xt, jnp.uint32)
    top = jnp.uint32(0x80000000)
    keys = jnp.where(u >= top, ~u, u | top)

    qs = jnp.linspace(0.0, 1.0, N_BINS + 1)[1:-1]
    q = qs * jnp.float32(N - 1)
    low = jnp.clip(jnp.floor(q), 0, N - 1)
    hw = (q - jnp.floor(q)).astype(jnp.float32)
    lw = (jnp.float32(1) - hw).astype(jnp.float32)
    jidx = low.astype(jnp.int32).reshape(NQ, 1)
    goff = jnp.repeat(jnp.arange(4, dtype=jnp.int32) * N_BINS,
                      jnp.array(GROUP_SIZES), total_repeat_length=C
                      ).reshape(C, 1)

    bins = pl.pallas_call(
        _bins_kernel,
        out_shape=jax.ShapeDtypeStruct((C, N), jnp.int32),
    )(keys, xt, jidx, lw.reshape(NQ, 1), hw.reshape(NQ, 1), goff)

    table = jnp.concatenate(
        [e.astype(jnp.float32) / jnp.float32(nf)
         for e, nf in zip((emb_global, emb_hour, emb_session,
                           emb_impression), GROUP_SIZES)], axis=0)
    table = jnp.pad(table, ((0, 0), (0, EMB_DIM)))

    sidx = (jnp.arange(16 * 4 * RCH, dtype=jnp.int32)
            .reshape(16, 4, RCH))
    out3 = _make_sc_pool()(table, bins, sidx)
    return out3[:, :, :EMB_DIM].transpose(1, 0, 2).reshape(N, 4 * EMB_DIM)


# speedup vs baseline: 1.9431x; 1.9431x over previous
import functools

import jax
import jax.numpy as jnp
from jax.experimental import pallas as pl

N = 16384
C = 16
N_BINS = 50
EMB_DIM = 64
GROUP_SIZES = (6, 3, 3, 4)
GROUP_STARTS = (0, 6, 9, 12)
NQ = N_BINS - 1

BCH = 8192
OCH = 2048


def _extract_kernel(keys_ref, x_ref, j_ref, lw_ref, hw_ref,
                    e0_ref, e1_ref, e2_ref, e3_ref, out_ref):
    j2 = j_ref[:]
    top = jnp.uint32(0x80000000)

    def bit_step(i, p):
        shift = (jnp.int32(31) - i).astype(jnp.uint32)
        trial = p | (jnp.uint32(1) << shift)

        def chunk_step(c, cnt):
            kb = keys_ref[:, pl.ds(c * BCH, BCH)]
            lt = kb[None, :, :] < trial[:, :, None]
            return cnt + jnp.sum(lt.astype(jnp.int32), axis=-1)

        cnt = jax.lax.fori_loop(
            0, N // BCH, chunk_step, jnp.zeros((NQ, C), jnp.int32))
        return jnp.where(cnt <= j2, trial, p)

    p = jax.lax.fori_loop(0, 32, bit_step, jnp.zeros((NQ, C), jnp.uint32))

    imax = jnp.int32(0x7FFFFFFF)

    def succ_step(c, carry):
        cle, mgt = carry
        kb = keys_ref[:, pl.ds(c * BCH, BCH)]
        kb3 = kb[None, :, :]
        le = kb3 <= p[:, :, None]
        cle = cle + jnp.sum(le.astype(jnp.int32), axis=-1)
        kb3_i = jax.lax.bitcast_convert_type(kb3 ^ top, jnp.int32)
        gtv = jnp.where(le, imax, kb3_i)
        mgt = jnp.minimum(mgt, jnp.min(gtv, axis=-1))
        return cle, mgt

    cle, mgt = jax.lax.fori_loop(
        0, N // BCH, succ_step,
        (jnp.zeros((NQ, C), jnp.int32), jnp.full((NQ, C), imax)))
    mgt_u = jax.lax.bitcast_convert_type(mgt, jnp.uint32) ^ top
    p_hi = jnp.where(cle >= j2 + 2, p, mgt_u)

    def unmap(k):
        u = jnp.where(k >= top, k & jnp.uint32(0x7FFFFFFF), ~k)
        return jax.lax.bitcast_convert_type(u, jnp.float32)

    lo_v = unmap(p)
    hi_v = unmap(p_hi)

    edges = lo_v * lw_ref[:] + hi_v * hw_ref[:]

    iota = jax.lax.broadcasted_iota(jnp.int32, (N_BINS, 1, 1), 0)
    embs = (e0_ref, e1_ref, e2_ref, e3_ref)

    def out_step(c, carry):
        xc = x_ref[:, pl.ds(c * OCH, OCH)]
        le = edges[:, :, None] <= xc[None, :, :]
        bins = jnp.sum(le.astype(jnp.int32), axis=0)
        for g in range(4):
            s = GROUP_STARTS[g]
            nf = GROUP_SIZES[g]
            bg = bins[s:s + nf, :]
            eq = (bg[None, :, :] == iota)
            a = jnp.sum(eq.astype(jnp.float32), axis=1)
            oc = jnp.dot(embs[g][:], a,
                         preferred_element_type=jnp.float32)
            out_ref[g * EMB_DIM:(g + 1) * EMB_DIM,
                    pl.ds(c * OCH, OCH)] = oc * jnp.float32(1.0 / nf)
        return carry

    jax.lax.fori_loop(0, N // OCH, out_step, jnp.int32(0))


@functools.partial(jax.jit)
def kernel(features, emb_global, emb_hour, emb_session, emb_impression):
    xt = features.T
    u = jax.lax.bitcast_convert_type(xt, jnp.uint32)
    top = jnp.uint32(0x80000000)
    keys = jnp.where(u >= top, ~u, u | top)

    qs = jnp.linspace(0.0, 1.0, N_BINS + 1)[1:-1]
    q = qs * jnp.float32(N - 1)
    low = jnp.clip(jnp.floor(q), 0, N - 1)
    hw = (q - jnp.floor(q)).astype(jnp.float32)
    lw = (jnp.float32(1) - hw).astype(jnp.float32)
    jidx = low.astype(jnp.int32).reshape(NQ, 1)

    et = [e.T.astype(jnp.float32) for e in
          (emb_global, emb_hour, emb_session, emb_impression)]

    out_t = pl.pallas_call(
        _extract_kernel,
        out_shape=jax.ShapeDtypeStruct((4 * EMB_DIM, N), jnp.float32),
    )(keys, xt, jidx, lw.reshape(NQ, 1), hw.reshape(NQ, 1), *et)
    return out_t.T


# speedup vs baseline: 1.9669x; 1.0122x over previous
import functools

import jax
import jax.numpy as jnp
from jax.experimental import pallas as pl

N = 16384
C = 16
N_BINS = 50
EMB_DIM = 64
GROUP_SIZES = (6, 3, 3, 4)
GROUP_STARTS = (0, 6, 9, 12)
NQ = N_BINS - 1

BCH = 8192
OCH = 2048


def _extract_kernel(keys_ref, x_ref, j_ref, lw_ref, hw_ref,
                    de0_ref, de1_ref, de2_ref, de3_ref, cv_ref, out_ref):
    j2 = j_ref[:]
    top = jnp.uint32(0x80000000)

    def bit_step(i, p):
        shift = (jnp.int32(31) - i).astype(jnp.uint32)
        trial = p | (jnp.uint32(1) << shift)

        def chunk_step(c, cnt):
            kb = keys_ref[:, pl.ds(c * BCH, BCH)]
            lt = kb[None, :, :] < trial[:, :, None]
            return cnt + jnp.sum(lt.astype(jnp.int32), axis=-1)

        cnt = jax.lax.fori_loop(
            0, N // BCH, chunk_step, jnp.zeros((NQ, C), jnp.int32))
        return jnp.where(cnt <= j2, trial, p)

    p = jax.lax.fori_loop(0, 32, bit_step, jnp.zeros((NQ, C), jnp.uint32))

    imax = jnp.int32(0x7FFFFFFF)

    def succ_step(c, carry):
        cle, mgt = carry
        kb = keys_ref[:, pl.ds(c * BCH, BCH)]
        kb3 = kb[None, :, :]
        le = kb3 <= p[:, :, None]
        cle = cle + jnp.sum(le.astype(jnp.int32), axis=-1)
        kb3_i = jax.lax.bitcast_convert_type(kb3 ^ top, jnp.int32)
        gtv = jnp.where(le, imax, kb3_i)
        mgt = jnp.minimum(mgt, jnp.min(gtv, axis=-1))
        return cle, mgt

    cle, mgt = jax.lax.fori_loop(
        0, N // BCH, succ_step,
        (jnp.zeros((NQ, C), jnp.int32), jnp.full((NQ, C), imax)))
    mgt_u = jax.lax.bitcast_convert_type(mgt, jnp.uint32) ^ top
    p_hi = jnp.where(cle >= j2 + 2, p, mgt_u)

    def unmap(k):
        u = jnp.where(k >= top, k & jnp.uint32(0x7FFFFFFF), ~k)
        return jax.lax.bitcast_convert_type(u, jnp.float32)

    lo_v = unmap(p)
    hi_v = unmap(p_hi)

    edges = lo_v * lw_ref[:] + hi_v * hw_ref[:]

    dembs = (de0_ref, de1_ref, de2_ref, de3_ref)
    cv = cv_ref[:]

    def out_step(c, carry):
        xc = x_ref[:, pl.ds(c * OCH, OCH)]
        le = (edges[:, :, None] <= xc[None, :, :]
              ).astype(jnp.float32)
        for g in range(4):
            s = GROUP_STARTS[g]
            nf = GROUP_SIZES[g]
            sg = jnp.sum(le[:, s:s + nf, :], axis=1)
            oc = jnp.dot(dembs[g][:], sg,
                         preferred_element_type=jnp.float32)
            out_ref[g * EMB_DIM:(g + 1) * EMB_DIM,
                    pl.ds(c * OCH, OCH)] = (
                oc + cv[g * EMB_DIM:(g + 1) * EMB_DIM, :])
        return carry

    jax.lax.fori_loop(0, N // OCH, out_step, jnp.int32(0))


@functools.partial(jax.jit)
def kernel(features, emb_global, emb_hour, emb_session, emb_impression):
    xt = features.T
    u = jax.lax.bitcast_convert_type(xt, jnp.uint32)
    top = jnp.uint32(0x80000000)
    keys = jnp.where(u >= top, ~u, u | top)

    qs = jnp.linspace(0.0, 1.0, N_BINS + 1)[1:-1]
    q = qs * jnp.float32(N - 1)
    low = jnp.clip(jnp.floor(q), 0, N - 1)
    hw = (q - jnp.floor(q)).astype(jnp.float32)
    lw = (jnp.float32(1) - hw).astype(jnp.float32)
    jidx = low.astype(jnp.int32).reshape(NQ, 1)

    es = (emb_global, emb_hour, emb_session, emb_impression)
    det = [((e[1:] - e[:-1]) / jnp.float32(nf)).T.astype(jnp.float32)
           for e, nf in zip(es, GROUP_SIZES)]
    cv = jnp.concatenate([e[0] for e in es]).reshape(4 * EMB_DIM, 1)

    out_t = pl.pallas_call(
        _extract_kernel,
        out_shape=jax.ShapeDtypeStruct((4 * EMB_DIM, N), jnp.float32),
    )(keys, xt, jidx, lw.reshape(NQ, 1), hw.reshape(NQ, 1), *det,
      cv.astype(jnp.float32))
    return out_t.T
